# initial kernel scaffold (unmeasured)
import jax
import jax.numpy as jnp
from jax import lax
from jax.experimental import pallas as pl
from jax.experimental.pallas import tpu as pltpu

N_DEV = 8
SQ = 1024
DH = 128
HPS = 8
HD = HPS * DH
SKV_SH = 1024
SKV = N_DEV * SKV_SH
KV1 = 1280
QG = 32
SCALE = 0.08838834764831843
RCH = SQ // N_DEV
BF = jnp.bfloat16
F32 = jnp.float32


def kernel(x, Wq, K_ext, V_ext, Wo):
    x2 = x.reshape(SQ, x.shape[-1])
    K2 = K_ext.reshape(SKV_SH, 64 * DH)
    V2 = V_ext.reshape(SKV_SH, 64 * DH)

    def body(x_ref, wq_ref, k_hbm, v_hbm, wo_ref, out_ref,
             kstage, vstage, ksend, vsend, kfull, vfull,
             ctx2d, psend, rbuf, gbuf, copy_sems, ssem, rsem):
        me = lax.axis_index("i")

        barrier = pltpu.get_barrier_semaphore()
        for k in range(1, N_DEV):
            p = (me + k) % N_DEV
            pl.semaphore_signal(barrier, inc=1, device_id=(p,),
                                device_id_type=pl.DeviceIdType.MESH)
        pl.semaphore_wait(barrier, N_DEV - 1)

        def stage(p):
            ck = pltpu.make_async_copy(
                k_hbm.at[:, pl.ds(p * HD, HD)], kstage, copy_sems.at[0])
            cv = pltpu.make_async_copy(
                v_hbm.at[:, pl.ds(p * HD, HD)], vstage, copy_sems.at[1])
            ck.start()
            cv.start()
            ck.wait()
            cv.wait()

        stage(me)
        kfull[pl.ds(me * SKV_SH, SKV_SH), :] = kstage[...].astype(BF)
        vfull[pl.ds(me * SKV_SH, SKV_SH), :] = vstage[...].astype(BF)

        kv_sends = []
        for k in range(1, N_DEV):
            p = (me + k) % N_DEV
            stage(p)
            ksend[k - 1] = kstage[...].astype(BF)
            vsend[k - 1] = vstage[...].astype(BF)
            rk = pltpu.make_async_remote_copy(
                src_ref=ksend.at[k - 1],
                dst_ref=kfull.at[pl.ds(me * SKV_SH, SKV_SH), :],
                send_sem=ssem.at[0, k - 1],
                recv_sem=rsem.at[0, me],
                device_id=(p,), device_id_type=pl.DeviceIdType.MESH)
            rv = pltpu.make_async_remote_copy(
                src_ref=vsend.at[k - 1],
                dst_ref=vfull.at[pl.ds(me * SKV_SH, SKV_SH), :],
                send_sem=ssem.at[1, k - 1],
                recv_sem=rsem.at[1, me],
                device_id=(p,), device_id_type=pl.DeviceIdType.MESH)
            rk.start()
            rv.start()
            kv_sends += [rk, rv]

        qm = jnp.dot(x_ref[...].astype(BF), wq_ref[...].astype(BF),
                     preferred_element_type=BF)

        for k in range(1, N_DEV):
            j = (me + k) % N_DEV
            for row, full in ((0, kfull), (1, vfull)):
                pltpu.make_async_remote_copy(
                    src_ref=ksend.at[0],
                    dst_ref=full.at[pl.ds(j * SKV_SH, SKV_SH), :],
                    send_sem=ssem.at[0, 0],
                    recv_sem=rsem.at[row, j],
                    device_id=(j,),
                    device_id_type=pl.DeviceIdType.MESH).wait_recv()

        qi1 = lax.broadcasted_iota(jnp.int32, (SQ, KV1), 0)
        ki1 = lax.broadcasted_iota(jnp.int32, (SQ, KV1), 1)
        mask1 = (jnp.abs(qi1 - ki1) <= 128) | (ki1 < QG)
        for h in range(HPS):
            c0 = h * DH
            qh = qm[:, c0:c0 + DH]
            s1 = lax.dot_general(
                qh, kfull[0:KV1, c0:c0 + DH],
                (((1,), (1,)), ((), ())),
                preferred_element_type=F32) * SCALE
            s1 = jnp.where(mask1, s1, -1e30)
            e1 = jnp.exp(s1 - jnp.max(s1, axis=1, keepdims=True))
            w1 = (e1 / jnp.sum(e1, axis=1, keepdims=True)).astype(BF)
            ctx1 = jnp.dot(w1, vfull[0:KV1, c0:c0 + DH],
                           preferred_element_type=F32)
            s2 = lax.dot_general(
                qh[0:QG, :], kfull[:, c0:c0 + DH],
                (((1,), (1,)), ((), ())),
                preferred_element_type=F32) * SCALE
            e2 = jnp.exp(s2 - jnp.max(s2, axis=1, keepdims=True))
            w2 = (e2 / jnp.sum(e2, axis=1, keepdims=True)).astype(BF)
            ctx2 = jnp.dot(w2, vfull[:, c0:c0 + DH],
                           preferred_element_type=F32)
            ctx2d[0:QG, c0:c0 + DH] = ctx2.astype(BF)
            ctx2d[QG:SQ, c0:c0 + DH] = ctx1[QG:SQ, :].astype(BF)

        psend[...] = jnp.dot(ctx2d[...], wo_ref[...].astype(BF),
                             preferred_element_type=BF)

        for d in kv_sends:
            d.wait_send()

        rs_sends = []
        for k in range(1, N_DEV):
            p = (me + k) % N_DEV
            rs = pltpu.make_async_remote_copy(
                src_ref=psend.at[pl.ds(p * RCH, RCH), :],
                dst_ref=rbuf.at[me],
                send_sem=ssem.at[0, k - 1],
                recv_sem=rsem.at[2, me],
                device_id=(p,), device_id_type=pl.DeviceIdType.MESH)
            rs.start()
            rs_sends.append(rs)
        rbuf[me] = psend[pl.ds(me * RCH, RCH), :]
        for k in range(1, N_DEV):
            j = (me + k) % N_DEV
            pltpu.make_async_remote_copy(
                src_ref=psend.at[pl.ds(0, RCH), :],
                dst_ref=rbuf.at[j],
                send_sem=ssem.at[0, 0],
                recv_sem=rsem.at[2, j],
                device_id=(j,),
                device_id_type=pl.DeviceIdType.MESH).wait_recv()
        gbuf[me] = jnp.sum(rbuf[...].astype(F32), axis=0).astype(BF)

        ag_sends = []
        for k in range(1, N_DEV):
            p = (me + k) % N_DEV
            ag = pltpu.make_async_remote_copy(
                src_ref=gbuf.at[me],
                dst_ref=gbuf.at[me],
                send_sem=ssem.at[1, k - 1],
                recv_sem=rsem.at[3, me],
                device_id=(p,), device_id_type=pl.DeviceIdType.MESH)
            ag.start()
            ag_sends.append(ag)
        for k in range(1, N_DEV):
            j = (me + k) % N_DEV
            pltpu.make_async_remote_copy(
                src_ref=gbuf.at[me],
                dst_ref=gbuf.at[j],
                send_sem=ssem.at[1, 0],
                recv_sem=rsem.at[3, j],
                device_id=(j,),
                device_id_type=pl.DeviceIdType.MESH).wait_recv()
        out_ref[...] = gbuf[...].reshape(SQ, SQ).astype(F32)

        for d in rs_sends + ag_sends:
            d.wait_send()

    out = pl.pallas_call(
        body,
        out_shape=jax.ShapeDtypeStruct((SQ, SQ), F32),
        in_specs=[
            pl.BlockSpec(memory_space=pltpu.VMEM),
            pl.BlockSpec(memory_space=pltpu.VMEM),
            pl.BlockSpec(memory_space=pltpu.ANY),
            pl.BlockSpec(memory_space=pltpu.ANY),
            pl.BlockSpec(memory_space=pltpu.VMEM),
        ],
        out_specs=pl.BlockSpec(memory_space=pltpu.VMEM),
        scratch_shapes=[
            pltpu.VMEM((SKV_SH, HD), F32),
            pltpu.VMEM((SKV_SH, HD), F32),
            pltpu.VMEM((N_DEV - 1, SKV_SH, HD), BF),
            pltpu.VMEM((N_DEV - 1, SKV_SH, HD), BF),
            pltpu.VMEM((SKV, HD), BF),
            pltpu.VMEM((SKV, HD), BF),
            pltpu.VMEM((SQ, HD), BF),
            pltpu.VMEM((SQ, SQ), BF),
            pltpu.VMEM((N_DEV, RCH, SQ), BF),
            pltpu.VMEM((N_DEV, RCH, SQ), BF),
            pltpu.SemaphoreType.DMA((2,)),
            pltpu.SemaphoreType.DMA((2, N_DEV - 1)),
            pltpu.SemaphoreType.DMA((4, N_DEV)),
        ],
        compiler_params=pltpu.CompilerParams(collective_id=0),
    )(x2, Wq, K2, V2, Wo)
    return out.reshape(1, SQ, SQ)


# baseline (device time: 478851 ns/iter reference)
import jax
import jax.numpy as jnp
from jax import lax
from jax.experimental import pallas as pl
from jax.experimental.pallas import tpu as pltpu

N_DEV = 8
SQ = 1024
DH = 128
HPS = 8
HD = HPS * DH
SKV_SH = 1024
SKV = N_DEV * SKV_SH
KV1 = 1280
QB = 512
QG = 32
SCALE = 0.08838834764831843
RCH = SQ // N_DEV
BF = jnp.bfloat16
F32 = jnp.float32
MESH = pl.DeviceIdType.MESH


def kernel(x, Wq, K_ext, V_ext, Wo):
    xb = x.reshape(SQ, x.shape[-1]).astype(BF)
    wqb = Wq.astype(BF)
    wob = Wo.astype(BF)
    k2 = K_ext.reshape(SKV_SH, N_DEV * HD).astype(BF)
    v2 = V_ext.reshape(SKV_SH, N_DEV * HD).astype(BF)

    def body(x_ref, wq_ref, k_hbm, v_hbm, wo_ref, out_ref, kfull, vfull,
             kband, vband, kc, vc,
             ctx2d, psend, rbuf, gbuf, copy_sems, ssem, rsem):
        me = lax.axis_index("i")

        barrier = pltpu.get_barrier_semaphore()
        for k in range(1, N_DEV):
            p = (me + k) % N_DEV
            pl.semaphore_signal(barrier, inc=1, device_id=(p,),
                                device_id_type=MESH)
        pl.semaphore_wait(barrier, N_DEV - 1)

        own_k = pltpu.make_async_copy(
            k_hbm.at[:, pl.ds(me * HD, HD)],
            kfull.at[pl.ds(me * SKV_SH, SKV_SH), :], copy_sems.at[0])
        own_v = pltpu.make_async_copy(
            v_hbm.at[:, pl.ds(me * HD, HD)],
            vfull.at[pl.ds(me * SKV_SH, SKV_SH), :], copy_sems.at[1])
        own_k.start()
        own_v.start()

        kv_sends = []
        for k in range(1, N_DEV):
            p = (me + k) % N_DEV
            rk = pltpu.make_async_remote_copy(
                src_ref=k_hbm.at[:, pl.ds(p * HD, HD)],
                dst_ref=kfull.at[pl.ds(me * SKV_SH, SKV_SH), :],
                send_sem=ssem.at[0, k - 1], recv_sem=rsem.at[0, me],
                device_id=(p,), device_id_type=MESH)
            rv = pltpu.make_async_remote_copy(
                src_ref=v_hbm.at[:, pl.ds(p * HD, HD)],
                dst_ref=vfull.at[pl.ds(me * SKV_SH, SKV_SH), :],
                send_sem=ssem.at[1, k - 1], recv_sem=rsem.at[1, me],
                device_id=(p,), device_id_type=MESH)
            rk.start()
            rv.start()
            kv_sends += [rk, rv]

        qm = jnp.dot(x_ref[...], wq_ref[...],
                     preferred_element_type=F32).astype(BF)

        own_k.wait()
        own_v.wait()
        for k in range(1, N_DEV):
            j = (me + k) % N_DEV
            for row, full in ((0, kfull), (1, vfull)):
                pltpu.make_async_remote_copy(
                    src_ref=k_hbm.at[:, pl.ds(0, HD)],
                    dst_ref=full.at[pl.ds(j * SKV_SH, SKV_SH), :],
                    send_sem=ssem.at[row, 0], recv_sem=rsem.at[row, j],
                    device_id=(j,), device_id_type=MESH).wait_recv()

        bk = pltpu.make_async_copy(
            kfull.at[pl.ds(0, KV1), :], kband, copy_sems.at[0])
        bv = pltpu.make_async_copy(
            vfull.at[pl.ds(0, KV1), :], vband, copy_sems.at[1])
        bk.start()
        bv.start()
        bk.wait()
        bv.wait()

        for h in range(HPS):
            c0 = h * DH
            for qb in range(SQ // QB):
                r0 = qb * QB
                qh = qm[r0:r0 + QB, c0:c0 + DH]
                s1 = lax.dot_general(
                    qh, kband[:, c0:c0 + DH],
                    (((1,), (1,)), ((), ())),
                    preferred_element_type=F32) * SCALE
                qi = r0 + lax.broadcasted_iota(jnp.int32, (QB, KV1), 0)
                ki = lax.broadcasted_iota(jnp.int32, (QB, KV1), 1)
                mask = (jnp.abs(qi - ki) <= 128) | (ki < QG)
                s1 = jnp.where(mask, s1, -1e30)
                e1 = jnp.exp(s1 - jnp.max(s1, axis=1, keepdims=True))
                w1 = (e1 / jnp.sum(e1, axis=1, keepdims=True)).astype(BF)
                ctx1 = jnp.dot(w1, vband[:, c0:c0 + DH],
                               preferred_element_type=F32)
                ctx2d[r0:r0 + QB, c0:c0 + DH] = ctx1.astype(BF)

        q2 = [qm[0:QG, h * DH:(h + 1) * DH] for h in range(HPS)]
        m_st = [jnp.full((QG, 1), -1e30, F32) for _ in range(HPS)]
        l_st = [jnp.zeros((QG, 1), F32) for _ in range(HPS)]
        a_st = [jnp.zeros((QG, DH), F32) for _ in range(HPS)]
        for c in range(N_DEV):
            ck = pltpu.make_async_copy(
                kfull.at[pl.ds(c * SKV_SH, SKV_SH), :], kc, copy_sems.at[0])
            cv = pltpu.make_async_copy(
                vfull.at[pl.ds(c * SKV_SH, SKV_SH), :], vc, copy_sems.at[1])
            ck.start()
            cv.start()
            ck.wait()
            cv.wait()
            for h in range(HPS):
                c0 = h * DH
                s = lax.dot_general(
                    q2[h], kc[:, c0:c0 + DH],
                    (((1,), (1,)), ((), ())),
                    preferred_element_type=F32) * SCALE
                m_new = jnp.maximum(m_st[h],
                                    jnp.max(s, axis=1, keepdims=True))
                alpha = jnp.exp(m_st[h] - m_new)
                pterm = jnp.exp(s - m_new)
                l_st[h] = l_st[h] * alpha + jnp.sum(pterm, axis=1,
                                                    keepdims=True)
                a_st[h] = a_st[h] * alpha + jnp.dot(
                    pterm.astype(BF), vc[:, c0:c0 + DH],
                    preferred_element_type=F32)
                m_st[h] = m_new
        for h in range(HPS):
            ctx2d[0:QG, h * DH:(h + 1) * DH] = (
                a_st[h] / l_st[h]).astype(BF)

        psend[...] = jnp.dot(ctx2d[...], wo_ref[...],
                             preferred_element_type=F32).astype(BF)

        for d in kv_sends:
            d.wait_send()

        rs_sends = []
        for k in range(1, N_DEV):
            p = (me + k) % N_DEV
            rs = pltpu.make_async_remote_copy(
                src_ref=psend.at[pl.ds(p * RCH, RCH), :],
                dst_ref=rbuf.at[me],
                send_sem=ssem.at[0, k - 1], recv_sem=rsem.at[2, me],
                device_id=(p,), device_id_type=MESH)
            rs.start()
            rs_sends.append(rs)
        rbuf[me] = psend[pl.ds(me * RCH, RCH), :]
        for k in range(1, N_DEV):
            j = (me + k) % N_DEV
            pltpu.make_async_remote_copy(
                src_ref=psend.at[pl.ds(0, RCH), :],
                dst_ref=rbuf.at[j],
                send_sem=ssem.at[0, 0], recv_sem=rsem.at[2, j],
                device_id=(j,), device_id_type=MESH).wait_recv()
        gbuf[me] = jnp.sum(rbuf[...].astype(F32), axis=0).astype(BF)

        ag_sends = []
        for k in range(1, N_DEV):
            p = (me + k) % N_DEV
            ag = pltpu.make_async_remote_copy(
                src_ref=gbuf.at[me],
                dst_ref=gbuf.at[me],
                send_sem=ssem.at[1, k - 1], recv_sem=rsem.at[3, me],
                device_id=(p,), device_id_type=MESH)
            ag.start()
            ag_sends.append(ag)
        for k in range(1, N_DEV):
            j = (me + k) % N_DEV
            pltpu.make_async_remote_copy(
                src_ref=gbuf.at[me],
                dst_ref=gbuf.at[j],
                send_sem=ssem.at[1, 0], recv_sem=rsem.at[3, j],
                device_id=(j,), device_id_type=MESH).wait_recv()
        out_ref[...] = gbuf[...].reshape(SQ, SQ).astype(F32)

        for d in rs_sends + ag_sends:
            d.wait_send()

    out = pl.pallas_call(
        body,
        out_shape=(
            jax.ShapeDtypeStruct((SQ, SQ), F32),
            jax.ShapeDtypeStruct((SKV, HD), BF),
            jax.ShapeDtypeStruct((SKV, HD), BF),
        ),
        in_specs=[
            pl.BlockSpec(memory_space=pltpu.VMEM),
            pl.BlockSpec(memory_space=pltpu.VMEM),
            pl.BlockSpec(memory_space=pl.ANY),
            pl.BlockSpec(memory_space=pl.ANY),
            pl.BlockSpec(memory_space=pltpu.VMEM),
        ],
        out_specs=(
            pl.BlockSpec(memory_space=pltpu.VMEM),
            pl.BlockSpec(memory_space=pl.ANY),
            pl.BlockSpec(memory_space=pl.ANY),
        ),
        scratch_shapes=[
            pltpu.VMEM((KV1, HD), BF),
            pltpu.VMEM((KV1, HD), BF),
            pltpu.VMEM((SKV_SH, HD), BF),
            pltpu.VMEM((SKV_SH, HD), BF),
            pltpu.VMEM((SQ, HD), BF),
            pltpu.VMEM((SQ, SQ), BF),
            pltpu.VMEM((N_DEV, RCH, SQ), BF),
            pltpu.VMEM((N_DEV, RCH, SQ), BF),
            pltpu.SemaphoreType.DMA((2,)),
            pltpu.SemaphoreType.DMA((2, N_DEV - 1)),
            pltpu.SemaphoreType.DMA((4, N_DEV)),
        ],
        compiler_params=pltpu.CompilerParams(collective_id=0),
    )(xb, wqb, k2, v2, wob)
    return out[0].reshape(1, SQ, SQ)


# device time: 454019 ns/iter; 1.0547x vs baseline; 1.0547x over previous
import jax
import jax.numpy as jnp
from jax import lax
from jax.experimental import pallas as pl
from jax.experimental.pallas import tpu as pltpu

N_DEV = 8
SQ = 1024
DH = 128
HPS = 8
HD = HPS * DH
SKV_SH = 1024
SKV = N_DEV * SKV_SH
KV1 = 1280
QB = 512
QG = 32
SCALE = 0.08838834764831843
RCH = SQ // N_DEV
NSLOT = 4
BF = jnp.bfloat16
F32 = jnp.float32
MESH = pl.DeviceIdType.MESH


def kernel(x, Wq, K_ext, V_ext, Wo):
    x2 = x.reshape(SQ, x.shape[-1])
    k2 = K_ext.reshape(SKV_SH, N_DEV * HD)
    v2 = V_ext.reshape(SKV_SH, N_DEV * HD)

    def body(x_ref, wq_ref, k_hbm, v_hbm, wo_ref, out_ref, kfull, vfull,
             kband, vband, kc, vc, fstage, sbuf,
             ctx2d, psend, rbuf, gbuf, copy_sems, ssem, rsem):
        me = lax.axis_index("i")

        barrier = pltpu.get_barrier_semaphore()
        for k in range(1, N_DEV):
            p = (me + k) % N_DEV
            pl.semaphore_signal(barrier, inc=1, device_id=(p,),
                                device_id_type=MESH)
        pl.semaphore_wait(barrier, N_DEV - 1)

        def stage_to(dst_ref, hbm, col, sem):
            cp = pltpu.make_async_copy(
                hbm.at[:, pl.ds(col, HD)], dst_ref, sem)
            cp.start()
            cp.wait()

        stage_to(fstage, k_hbm, me * HD, copy_sems.at[0])
        kc[...] = fstage[...].astype(BF)
        own_k = pltpu.make_async_copy(
            kc, kfull.at[pl.ds(me * SKV_SH, SKV_SH), :], copy_sems.at[2])
        own_k.start()
        stage_to(fstage, v_hbm, me * HD, copy_sems.at[1])
        vc[...] = fstage[...].astype(BF)
        own_v = pltpu.make_async_copy(
            vc, vfull.at[pl.ds(me * SKV_SH, SKV_SH), :], copy_sems.at[3])
        own_v.start()

        sends = []
        for k in range(1, N_DEV):
            p = (me + k) % N_DEV
            for row, hbm, full in ((0, k_hbm, kfull), (1, v_hbm, vfull)):
                cnt = 2 * (k - 1) + row
                s = cnt % NSLOT
                if cnt >= NSLOT:
                    sends[cnt - NSLOT].wait_send()
                stage_to(fstage, hbm, p * HD, copy_sems.at[row])
                sbuf[s] = fstage[...].astype(BF)
                r = pltpu.make_async_remote_copy(
                    src_ref=sbuf.at[s],
                    dst_ref=full.at[pl.ds(me * SKV_SH, SKV_SH), :],
                    send_sem=ssem.at[row, k - 1], recv_sem=rsem.at[row, me],
                    device_id=(p,), device_id_type=MESH)
                r.start()
                sends.append(r)

        qm = jnp.dot(x_ref[...].astype(BF), wq_ref[...].astype(BF),
                     preferred_element_type=F32).astype(BF)

        own_k.wait()
        own_v.wait()

        def wait_from(j, row, full):
            pltpu.make_async_remote_copy(
                src_ref=sbuf.at[0],
                dst_ref=full.at[pl.ds(j * SKV_SH, SKV_SH), :],
                send_sem=ssem.at[row, 0], recv_sem=rsem.at[row, j],
                device_id=(j,), device_id_type=MESH).wait_recv()

        for j in (0, 1):
            @pl.when(me != j)
            def _(j=j):
                wait_from(j, 0, kfull)
                wait_from(j, 1, vfull)

        bk = pltpu.make_async_copy(
            kfull.at[pl.ds(0, KV1), :], kband, copy_sems.at[0])
        bv = pltpu.make_async_copy(
            vfull.at[pl.ds(0, KV1), :], vband, copy_sems.at[1])
        bk.start()
        bv.start()
        bk.wait()
        bv.wait()

        for qb in range(SQ // QB):
            r0 = qb * QB
            qi = r0 + lax.broadcasted_iota(jnp.int32, (QB, KV1), 0)
            ki = lax.broadcasted_iota(jnp.int32, (QB, KV1), 1)
            mask = (jnp.abs(qi - ki) <= 128) | (ki < QG)
            madd = jnp.where(mask, 0.0, -1e30)
            for h in range(HPS):
                c0 = h * DH
                qh = qm[r0:r0 + QB, c0:c0 + DH]
                s1 = lax.dot_general(
                    qh, kband[:, c0:c0 + DH],
                    (((1,), (1,)), ((), ())),
                    preferred_element_type=F32) * SCALE + madd
                e1 = jnp.exp(s1 - jnp.max(s1, axis=1, keepdims=True))
                w1 = (e1 * (1.0 / jnp.sum(e1, axis=1, keepdims=True))
                      ).astype(BF)
                ctx1 = jnp.dot(w1, vband[:, c0:c0 + DH],
                               preferred_element_type=F32)
                ctx2d[r0:r0 + QB, c0:c0 + DH] = ctx1.astype(BF)

        for k in range(1, N_DEV):
            j = (me + k) % N_DEV

            @pl.when(jnp.logical_and(j != 0, j != 1))
            def _(j=j):
                wait_from(j, 0, kfull)
                wait_from(j, 1, vfull)

        q2 = [qm[0:QG, h * DH:(h + 1) * DH] for h in range(HPS)]
        m_st = [jnp.full((QG, 1), -1e30, F32) for _ in range(HPS)]
        l_st = [jnp.zeros((QG, 1), F32) for _ in range(HPS)]
        a_st = [jnp.zeros((QG, DH), F32) for _ in range(HPS)]

        def p2_update(h, kk, vv):
            c0 = h * DH
            s = lax.dot_general(
                q2[h], kk[:, c0:c0 + DH],
                (((1,), (1,)), ((), ())),
                preferred_element_type=F32) * SCALE
            m_new = jnp.maximum(m_st[h], jnp.max(s, axis=1, keepdims=True))
            alpha = jnp.exp(m_st[h] - m_new)
            pterm = jnp.exp(s - m_new)
            l_st[h] = l_st[h] * alpha + jnp.sum(pterm, axis=1, keepdims=True)
            a_st[h] = a_st[h] * alpha + jnp.dot(
                pterm.astype(BF), vv[:, c0:c0 + DH],
                preferred_element_type=F32)
            m_st[h] = m_new

        for h in range(HPS):
            p2_update(h, kband[0:SKV_SH, :], vband[0:SKV_SH, :])
        for c in range(1, N_DEV):
            ck = pltpu.make_async_copy(
                kfull.at[pl.ds(c * SKV_SH, SKV_SH), :], kc, copy_sems.at[0])
            cv = pltpu.make_async_copy(
                vfull.at[pl.ds(c * SKV_SH, SKV_SH), :], vc, copy_sems.at[1])
            ck.start()
            cv.start()
            ck.wait()
            cv.wait()
            for h in range(HPS):
                p2_update(h, kc[...], vc[...])
        for h in range(HPS):
            ctx2d[0:QG, h * DH:(h + 1) * DH] = (
                a_st[h] * (1.0 / l_st[h])).astype(BF)

        psend[...] = jnp.dot(ctx2d[...], wo_ref[...].astype(BF),
                             preferred_element_type=F32).astype(BF)

        for d in sends[-NSLOT:]:
            d.wait_send()

        rs_sends = []
        for k in range(1, N_DEV):
            p = (me + k) % N_DEV
            rs = pltpu.make_async_remote_copy(
                src_ref=psend.at[pl.ds(p * RCH, RCH), :],
                dst_ref=rbuf.at[me],
                send_sem=ssem.at[0, k - 1], recv_sem=rsem.at[2, me],
                device_id=(p,), device_id_type=MESH)
            rs.start()
            rs_sends.append(rs)
        rbuf[me] = psend[pl.ds(me * RCH, RCH), :]
        for k in range(1, N_DEV):
            j = (me + k) % N_DEV
            pltpu.make_async_remote_copy(
                src_ref=psend.at[pl.ds(0, RCH), :],
                dst_ref=rbuf.at[j],
                send_sem=ssem.at[0, 0], recv_sem=rsem.at[2, j],
                device_id=(j,), device_id_type=MESH).wait_recv()
        gbuf[me] = jnp.sum(rbuf[...].astype(F32), axis=0).astype(BF)

        ag_sends = []
        for k in range(1, N_DEV):
            p = (me + k) % N_DEV
            ag = pltpu.make_async_remote_copy(
                src_ref=gbuf.at[me],
                dst_ref=gbuf.at[me],
                send_sem=ssem.at[1, k - 1], recv_sem=rsem.at[3, me],
                device_id=(p,), device_id_type=MESH)
            ag.start()
            ag_sends.append(ag)
        for k in range(1, N_DEV):
            j = (me + k) % N_DEV
            pltpu.make_async_remote_copy(
                src_ref=gbuf.at[me],
                dst_ref=gbuf.at[j],
                send_sem=ssem.at[1, 0], recv_sem=rsem.at[3, j],
                device_id=(j,), device_id_type=MESH).wait_recv()
        out_ref[...] = gbuf[...].reshape(SQ, SQ).astype(F32)

        for d in rs_sends + ag_sends:
            d.wait_send()

    out = pl.pallas_call(
        body,
        out_shape=(
            jax.ShapeDtypeStruct((SQ, SQ), F32),
            jax.ShapeDtypeStruct((SKV, HD), BF),
            jax.ShapeDtypeStruct((SKV, HD), BF),
        ),
        in_specs=[
            pl.BlockSpec(memory_space=pltpu.VMEM),
            pl.BlockSpec(memory_space=pltpu.VMEM),
            pl.BlockSpec(memory_space=pl.ANY),
            pl.BlockSpec(memory_space=pl.ANY),
            pl.BlockSpec(memory_space=pltpu.VMEM),
        ],
        out_specs=(
            pl.BlockSpec(memory_space=pltpu.VMEM),
            pl.BlockSpec(memory_space=pl.ANY),
            pl.BlockSpec(memory_space=pl.ANY),
        ),
        scratch_shapes=[
            pltpu.VMEM((KV1, HD), BF),
            pltpu.VMEM((KV1, HD), BF),
            pltpu.VMEM((SKV_SH, HD), BF),
            pltpu.VMEM((SKV_SH, HD), BF),
            pltpu.VMEM((SKV_SH, HD), F32),
            pltpu.VMEM((NSLOT, SKV_SH, HD), BF),
            pltpu.VMEM((SQ, HD), BF),
            pltpu.VMEM((SQ, SQ), BF),
            pltpu.VMEM((N_DEV, RCH, SQ), BF),
            pltpu.VMEM((N_DEV, RCH, SQ), BF),
            pltpu.SemaphoreType.DMA((4,)),
            pltpu.SemaphoreType.DMA((2, N_DEV - 1)),
            pltpu.SemaphoreType.DMA((4, N_DEV)),
        ],
        compiler_params=pltpu.CompilerParams(
            collective_id=0, vmem_limit_bytes=58 * 1024 * 1024),
    )(x2, Wq, k2, v2, Wo)
    return out[0].reshape(1, SQ, SQ)


# device time: 346659 ns/iter; 1.3813x vs baseline; 1.3097x over previous
import jax
import jax.numpy as jnp
from jax import lax
from jax.experimental import pallas as pl
from jax.experimental.pallas import tpu as pltpu

N_DEV = 8
SQ = 1024
DH = 128
HPS = 8
HD = HPS * DH
SKV_SH = 1024
SKV = N_DEV * SKV_SH
KV1 = 1280
B1R = 256
QB = 512
QG = 32
SCALE = 0.08838834764831843
RCH = SQ // N_DEV
BF = jnp.bfloat16
F32 = jnp.float32
MESH = pl.DeviceIdType.MESH


def kernel(x, Wq, K_ext, V_ext, Wo):
    def body(x_ref, wq_ref, k_hbm, v_hbm, wo_ref, out_ref,
             q32buf, fstage, sbk, sbv, kband, vband,
             sacc_snd, sml_snd, racc, rml,
             ctx2d, psend, rbuf, gbuf, copy_sems, ssem, rsem):
        me = lax.axis_index("i")

        barrier = pltpu.get_barrier_semaphore()
        for k in range(1, N_DEV):
            p = (me + k) % N_DEV
            pl.semaphore_signal(barrier, inc=1, device_id=(p,),
                                device_id_type=MESH)
        pl.semaphore_wait(barrier, N_DEV - 1)

        qm = jnp.dot(x_ref[0].astype(BF), wq_ref[...].astype(BF),
                     preferred_element_type=F32).astype(BF)
        q32buf[me] = qm[0:QG, :]
        q32_sends = []
        for k in range(1, N_DEV):
            p = (me + k) % N_DEV
            r = pltpu.make_async_remote_copy(
                src_ref=q32buf.at[me], dst_ref=q32buf.at[me],
                send_sem=ssem.at[0, k - 1], recv_sem=rsem.at[0, me],
                device_id=(p,), device_id_type=MESH)
            r.start()
            q32_sends.append(r)
        for k in range(1, N_DEV):
            j = (me + k) % N_DEV
            pltpu.make_async_remote_copy(
                src_ref=q32buf.at[0], dst_ref=q32buf.at[j],
                send_sem=ssem.at[0, 0], recv_sem=rsem.at[0, j],
                device_id=(j,), device_id_type=MESH).wait_recv()
        for r in q32_sends:
            r.wait_send()

        d0 = {}
        d1 = {}
        for k in range(N_DEV):
            p = (me + k) % N_DEV
            slot = k % 2
            if k >= 3:
                @pl.when(me == 0)
                def _(kk=k - 2):
                    for d in d0[kk]:
                        d.wait_send()

                @pl.when(me == 1)
                def _(kk=k - 2):
                    for d in d1[kk]:
                        d.wait_send()
            for hbm, sb, sem in ((k_hbm, sbk, 0), (v_hbm, sbv, 1)):
                cp = pltpu.make_async_copy(
                    hbm.at[0, :, pl.ds(p * HPS, HPS), :], fstage,
                    copy_sems.at[sem])
                cp.start()
                cp.wait()
                for hh in range(HPS):
                    sb[slot, :, hh * DH:(hh + 1) * DH] = (
                        fstage[:, hh, :].astype(BF))
            ms, ls = [], []
            for hh in range(HPS):
                c0 = hh * DH
                s = lax.dot_general(
                    q32buf[p, :, c0:c0 + DH], sbk[slot, :, c0:c0 + DH],
                    (((1,), (1,)), ((), ())),
                    preferred_element_type=F32) * SCALE
                m = jnp.max(s, axis=1, keepdims=True)
                e = jnp.exp(s - m)
                ls.append(jnp.sum(e, axis=1, keepdims=True))
                ms.append(m)
                sacc_snd[k, :, c0:c0 + DH] = jnp.dot(
                    e.astype(BF), sbv[slot, :, c0:c0 + DH],
                    preferred_element_type=F32).astype(BF)
            sml_snd[k, :, 0:2 * HPS] = jnp.concatenate(ms + ls, axis=1)
            if k == 0:
                racc[me] = sacc_snd[0]
                rml[me] = sml_snd[0]

                @pl.when(me == 0)
                def _():
                    kband[0:SKV_SH, :] = sbk[0]
                    vband[0:SKV_SH, :] = sbv[0]

                @pl.when(me == 1)
                def _():
                    kband[SKV_SH:KV1, :] = sbk[0, 0:B1R, :]
                    vband[SKV_SH:KV1, :] = sbv[0, 0:B1R, :]
            else:
                b0 = [pltpu.make_async_remote_copy(
                    src_ref=sb.at[slot],
                    dst_ref=bd.at[pl.ds(0, SKV_SH), :],
                    send_sem=ssem.at[row, k - 1], recv_sem=rsem.at[1 + row, me],
                    device_id=(p,), device_id_type=MESH)
                    for row, sb, bd in ((0, sbk, kband), (1, sbv, vband))]
                b1 = [pltpu.make_async_remote_copy(
                    src_ref=sb.at[slot, pl.ds(0, B1R), :],
                    dst_ref=bd.at[pl.ds(SKV_SH, B1R), :],
                    send_sem=ssem.at[row, k - 1], recv_sem=rsem.at[1 + row, me],
                    device_id=(p,), device_id_type=MESH)
                    for row, sb, bd in ((0, sbk, kband), (1, sbv, vband))]
                d0[k] = b0
                d1[k] = b1

                @pl.when(me == 0)
                def _():
                    for d in b0:
                        d.start()

                @pl.when(me == 1)
                def _():
                    for d in b1:
                        d.start()
                for row, src, dst in ((2, sacc_snd, racc), (3, sml_snd, rml)):
                    pltpu.make_async_remote_copy(
                        src_ref=src.at[k], dst_ref=dst.at[me],
                        send_sem=ssem.at[row, k - 1],
                        recv_sem=rsem.at[1 + row, me],
                        device_id=(p,), device_id_type=MESH).start()

        for src_shard, rows in ((0, (0, SKV_SH)), (1, (SKV_SH, B1R))):
            @pl.when(me != src_shard)
            def _(src_shard=src_shard, rows=rows):
                for row, bd in ((0, kband), (1, vband)):
                    pltpu.make_async_remote_copy(
                        src_ref=bd.at[pl.ds(0, rows[1]), :],
                        dst_ref=bd.at[pl.ds(rows[0], rows[1]), :],
                        send_sem=ssem.at[row, 0],
                        recv_sem=rsem.at[1 + row, src_shard],
                        device_id=(src_shard,),
                        device_id_type=MESH).wait_recv()

        for qb in range(SQ // QB):
            r0 = qb * QB
            qi = r0 + lax.broadcasted_iota(jnp.int32, (QB, KV1), 0)
            ki = lax.broadcasted_iota(jnp.int32, (QB, KV1), 1)
            madd = jnp.where(
                (jnp.abs(qi - ki) <= 128) | (ki < QG), 0.0, -1e30)
            for h in range(HPS):
                c0 = h * DH
                s1 = lax.dot_general(
                    qm[r0:r0 + QB, c0:c0 + DH], kband[:, c0:c0 + DH],
                    (((1,), (1,)), ((), ())),
                    preferred_element_type=F32) * SCALE + madd
                e1 = jnp.exp(s1 - jnp.max(s1, axis=1, keepdims=True))
                w1 = (e1 * (1.0 / jnp.sum(e1, axis=1, keepdims=True))
                      ).astype(BF)
                ctx1 = jnp.dot(w1, vband[:, c0:c0 + DH],
                               preferred_element_type=F32)
                ctx2d[r0:r0 + QB, c0:c0 + DH] = ctx1.astype(BF)

        for k in range(1, N_DEV):
            j = (me + k) % N_DEV
            for row, src, dst in ((2, sacc_snd, racc), (3, sml_snd, rml)):
                pltpu.make_async_remote_copy(
                    src_ref=src.at[0], dst_ref=dst.at[j],
                    send_sem=ssem.at[row, 0], recv_sem=rsem.at[1 + row, j],
                    device_id=(j,), device_id_type=MESH).wait_recv()
        R = rml[...]
        A = racc[...]
        for h in range(HPS):
            mj = R[:, :, h]
            lj = R[:, :, HPS + h]
            m_g = jnp.max(mj, axis=0, keepdims=True)
            alpha = jnp.exp(mj - m_g)
            l_g = jnp.sum(lj * alpha, axis=0, keepdims=True)
            accj = A[:, :, h * DH:(h + 1) * DH].astype(F32)
            acc_g = jnp.sum(accj * alpha[:, :, None], axis=0)
            inv = jnp.reshape(1.0 / l_g, (QG, 1))
            ctx2d[0:QG, h * DH:(h + 1) * DH] = (acc_g * inv).astype(BF)

        psend[...] = jnp.dot(ctx2d[...], wo_ref[...].astype(BF),
                             preferred_element_type=F32).astype(BF)

        @pl.when(me == 0)
        def _():
            for kk in (N_DEV - 2, N_DEV - 1):
                for d in d0[kk]:
                    d.wait_send()

        @pl.when(me == 1)
        def _():
            for kk in (N_DEV - 2, N_DEV - 1):
                for d in d1[kk]:
                    d.wait_send()

        rs_sends = []
        for k in range(1, N_DEV):
            p = (me + k) % N_DEV
            rs = pltpu.make_async_remote_copy(
                src_ref=psend.at[pl.ds(p * RCH, RCH), :],
                dst_ref=rbuf.at[me],
                send_sem=ssem.at[0, k - 1], recv_sem=rsem.at[6, me],
                device_id=(p,), device_id_type=MESH)
            rs.start()
            rs_sends.append(rs)
        rbuf[me] = psend[pl.ds(me * RCH, RCH), :]
        for k in range(1, N_DEV):
            j = (me + k) % N_DEV
            pltpu.make_async_remote_copy(
                src_ref=psend.at[pl.ds(0, RCH), :],
                dst_ref=rbuf.at[j],
                send_sem=ssem.at[0, 0], recv_sem=rsem.at[6, j],
                device_id=(j,), device_id_type=MESH).wait_recv()
        gbuf[me] = jnp.sum(rbuf[...].astype(F32), axis=0).astype(BF)

        ag_sends = []
        for k in range(1, N_DEV):
            p = (me + k) % N_DEV
            ag = pltpu.make_async_remote_copy(
                src_ref=gbuf.at[me], dst_ref=gbuf.at[me],
                send_sem=ssem.at[1, k - 1], recv_sem=rsem.at[7, me],
                device_id=(p,), device_id_type=MESH)
            ag.start()
            ag_sends.append(ag)
        for k in range(1, N_DEV):
            j = (me + k) % N_DEV
            pltpu.make_async_remote_copy(
                src_ref=gbuf.at[me], dst_ref=gbuf.at[j],
                send_sem=ssem.at[1, 0], recv_sem=rsem.at[7, j],
                device_id=(j,), device_id_type=MESH).wait_recv()
        out_ref[...] = gbuf[...].reshape(SQ, SQ).astype(F32)

        for k in range(1, N_DEV):
            for row in (2, 3):
                pltpu.make_async_remote_copy(
                    src_ref=sacc_snd.at[0] if row == 2 else sml_snd.at[0],
                    dst_ref=racc.at[0] if row == 2 else rml.at[0],
                    send_sem=ssem.at[row, k - 1], recv_sem=rsem.at[1 + row, 0],
                    device_id=(0,), device_id_type=MESH).wait_send()
        for d in rs_sends + ag_sends:
            d.wait_send()

    out = pl.pallas_call(
        body,
        out_shape=jax.ShapeDtypeStruct((SQ, SQ), F32),
        in_specs=[
            pl.BlockSpec(memory_space=pltpu.VMEM),
            pl.BlockSpec(memory_space=pltpu.VMEM),
            pl.BlockSpec(memory_space=pl.ANY),
            pl.BlockSpec(memory_space=pl.ANY),
            pl.BlockSpec(memory_space=pltpu.VMEM),
        ],
        out_specs=pl.BlockSpec(memory_space=pltpu.VMEM),
        scratch_shapes=[
            pltpu.VMEM((N_DEV, QG, HD), BF),
            pltpu.VMEM((SKV_SH, HPS, DH), F32),
            pltpu.VMEM((2, SKV_SH, HD), BF),
            pltpu.VMEM((2, SKV_SH, HD), BF),
            pltpu.VMEM((KV1, HD), BF),
            pltpu.VMEM((KV1, HD), BF),
            pltpu.VMEM((N_DEV, QG, HD), BF),
            pltpu.VMEM((N_DEV, QG, DH), F32),
            pltpu.VMEM((N_DEV, QG, HD), BF),
            pltpu.VMEM((N_DEV, QG, DH), F32),
            pltpu.VMEM((SQ, HD), BF),
            pltpu.VMEM((SQ, SQ), BF),
            pltpu.VMEM((N_DEV, RCH, SQ), BF),
            pltpu.VMEM((N_DEV, RCH, SQ), BF),
            pltpu.SemaphoreType.DMA((2,)),
            pltpu.SemaphoreType.DMA((4, N_DEV - 1)),
            pltpu.SemaphoreType.DMA((8, N_DEV)),
        ],
        compiler_params=pltpu.CompilerParams(
            collective_id=0, vmem_limit_bytes=58 * 1024 * 1024),
    )(x, Wq, K_ext, V_ext, Wo)
    return out.reshape(1, SQ, SQ)


# device time: 307098 ns/iter; 1.5593x vs baseline; 1.1288x over previous
import jax
import jax.numpy as jnp
from jax import lax
from jax.experimental import pallas as pl
from jax.experimental.pallas import tpu as pltpu

N_DEV = 8
SQ = 1024
DH = 128
HPS = 8
HD = HPS * DH
SKV_SH = 1024
SKV = N_DEV * SKV_SH
KV1 = 1280
B1R = 256
QB = 512
QG = 32
SCALE = 0.08838834764831843
RCH = SQ // N_DEV
BF = jnp.bfloat16
F32 = jnp.float32
MESH = pl.DeviceIdType.MESH


def kernel(x, Wq, K_ext, V_ext, Wo):
    def body(x_ref, wq_ref, k_hbm, v_hbm, wo_ref, out_ref,
             q32buf, fstage, sbk, sbv, kband, vband,
             sacc_snd, sml_snd, racc, rml,
             ctx2d, psend, rbuf, gbuf, copy_sems, ssem, rsem):
        me = lax.axis_index("i")

        barrier = pltpu.get_barrier_semaphore()
        for k in range(1, N_DEV):
            p = (me + k) % N_DEV
            pl.semaphore_signal(barrier, inc=1, device_id=(p,),
                                device_id_type=MESH)
        pl.semaphore_wait(barrier, N_DEV - 1)

        qm = jnp.dot(x_ref[0].astype(BF), wq_ref[...].astype(BF),
                     preferred_element_type=F32).astype(BF)
        q32buf[me] = qm[0:QG, :]
        q32_sends = []
        for k in range(1, N_DEV):
            p = (me + k) % N_DEV
            r = pltpu.make_async_remote_copy(
                src_ref=q32buf.at[me], dst_ref=q32buf.at[me],
                send_sem=ssem.at[0, k - 1], recv_sem=rsem.at[0, me],
                device_id=(p,), device_id_type=MESH)
            r.start()
            q32_sends.append(r)
        for k in range(1, N_DEV):
            j = (me + k) % N_DEV
            pltpu.make_async_remote_copy(
                src_ref=q32buf.at[0], dst_ref=q32buf.at[j],
                send_sem=ssem.at[0, 0], recv_sem=rsem.at[0, j],
                device_id=(j,), device_id_type=MESH).wait_recv()
        for r in q32_sends:
            r.wait_send()

        d0 = {}
        d1 = {}
        for k in range(N_DEV):
            p = (me + k) % N_DEV
            slot = k % 3
            if k >= 4:
                @pl.when(me == 0)
                def _(kk=k - 3):
                    for d in d0[kk]:
                        d.wait_send()

                @pl.when(me == 1)
                def _(kk=k - 3):
                    for d in d1[kk]:
                        d.wait_send()
            for hbm, sb, sem in ((k_hbm, sbk, 0), (v_hbm, sbv, 1)):
                cp = pltpu.make_async_copy(
                    hbm.at[0, :, pl.ds(p * HPS, HPS), :], fstage,
                    copy_sems.at[sem])
                cp.start()
                cp.wait()
                for hh in range(HPS):
                    sb[slot, :, hh * DH:(hh + 1) * DH] = (
                        fstage[:, hh, :].astype(BF))
            ms, ls = [], []
            for hh in range(HPS):
                c0 = hh * DH
                s = lax.dot_general(
                    q32buf[p, :, c0:c0 + DH], sbk[slot, :, c0:c0 + DH],
                    (((1,), (1,)), ((), ())),
                    preferred_element_type=F32) * SCALE
                m = jnp.max(s, axis=1, keepdims=True)
                e = jnp.exp(s - m)
                ls.append(jnp.sum(e, axis=1, keepdims=True))
                ms.append(m)
                sacc_snd[k, :, c0:c0 + DH] = jnp.dot(
                    e.astype(BF), sbv[slot, :, c0:c0 + DH],
                    preferred_element_type=F32).astype(BF)
            sml_snd[k, :, 0:2 * HPS] = jnp.concatenate(ms + ls, axis=1)
            if k == 0:
                racc[me] = sacc_snd[0]
                rml[me] = sml_snd[0]

                @pl.when(me == 0)
                def _():
                    kband[0:SKV_SH, :] = sbk[0]
                    vband[0:SKV_SH, :] = sbv[0]

                @pl.when(me == 1)
                def _():
                    kband[SKV_SH:KV1, :] = sbk[0, 0:B1R, :]
                    vband[SKV_SH:KV1, :] = sbv[0, 0:B1R, :]
            else:
                b0 = [pltpu.make_async_remote_copy(
                    src_ref=sb.at[slot],
                    dst_ref=bd.at[pl.ds(0, SKV_SH), :],
                    send_sem=ssem.at[row, k - 1], recv_sem=rsem.at[1 + row, me],
                    device_id=(p,), device_id_type=MESH)
                    for row, sb, bd in ((0, sbk, kband), (1, sbv, vband))]
                b1 = [pltpu.make_async_remote_copy(
                    src_ref=sb.at[slot, pl.ds(0, B1R), :],
                    dst_ref=bd.at[pl.ds(SKV_SH, B1R), :],
                    send_sem=ssem.at[row, k - 1], recv_sem=rsem.at[1 + row, me],
                    device_id=(p,), device_id_type=MESH)
                    for row, sb, bd in ((0, sbk, kband), (1, sbv, vband))]
                d0[k] = b0
                d1[k] = b1

                @pl.when(me == 0)
                def _():
                    for d in b0:
                        d.start()

                @pl.when(me == 1)
                def _():
                    for d in b1:
                        d.start()
                for row, src, dst in ((2, sacc_snd, racc), (3, sml_snd, rml)):
                    pltpu.make_async_remote_copy(
                        src_ref=src.at[k], dst_ref=dst.at[me],
                        send_sem=ssem.at[row, k - 1],
                        recv_sem=rsem.at[1 + row, me],
                        device_id=(p,), device_id_type=MESH).start()

        for src_shard, rows in ((0, (0, SKV_SH)), (1, (SKV_SH, B1R))):
            @pl.when(me != src_shard)
            def _(src_shard=src_shard, rows=rows):
                for row, bd in ((0, kband), (1, vband)):
                    pltpu.make_async_remote_copy(
                        src_ref=bd.at[pl.ds(0, rows[1]), :],
                        dst_ref=bd.at[pl.ds(rows[0], rows[1]), :],
                        send_sem=ssem.at[row, 0],
                        recv_sem=rsem.at[1 + row, src_shard],
                        device_id=(src_shard,),
                        device_id_type=MESH).wait_recv()

        for qb, ext in ((0, 640), (1, 1152)):
            r0 = qb * QB
            qi = r0 + lax.broadcasted_iota(jnp.int32, (QB, ext), 0)
            ki = lax.broadcasted_iota(jnp.int32, (QB, ext), 1)
            madd = jnp.where(
                (jnp.abs(qi - ki) <= 128) | (ki < QG), 0.0, -1e30)
            for h in range(HPS):
                c0 = h * DH
                s1 = lax.dot_general(
                    qm[r0:r0 + QB, c0:c0 + DH], kband[0:ext, c0:c0 + DH],
                    (((1,), (1,)), ((), ())),
                    preferred_element_type=F32) * SCALE + madd
                e1 = jnp.exp(s1 - jnp.max(s1, axis=1, keepdims=True))
                w1 = (e1 * (1.0 / jnp.sum(e1, axis=1, keepdims=True))
                      ).astype(BF)
                ctx1 = jnp.dot(w1, vband[0:ext, c0:c0 + DH],
                               preferred_element_type=F32)
                ctx2d[r0:r0 + QB, c0:c0 + DH] = ctx1.astype(BF)

        for k in range(1, N_DEV):
            j = (me + k) % N_DEV
            for row, src, dst in ((2, sacc_snd, racc), (3, sml_snd, rml)):
                pltpu.make_async_remote_copy(
                    src_ref=src.at[0], dst_ref=dst.at[j],
                    send_sem=ssem.at[row, 0], recv_sem=rsem.at[1 + row, j],
                    device_id=(j,), device_id_type=MESH).wait_recv()
        R = rml[...]
        A = racc[...]
        for h in range(HPS):
            mj = R[:, :, h]
            lj = R[:, :, HPS + h]
            m_g = jnp.max(mj, axis=0, keepdims=True)
            alpha = jnp.exp(mj - m_g)
            l_g = jnp.sum(lj * alpha, axis=0, keepdims=True)
            accj = A[:, :, h * DH:(h + 1) * DH].astype(F32)
            acc_g = jnp.sum(accj * alpha[:, :, None], axis=0)
            inv = jnp.reshape(1.0 / l_g, (QG, 1))
            ctx2d[0:QG, h * DH:(h + 1) * DH] = (acc_g * inv).astype(BF)

        psend[...] = jnp.dot(ctx2d[...], wo_ref[...].astype(BF),
                             preferred_element_type=F32).astype(BF)

        @pl.when(me == 0)
        def _():
            for kk in (N_DEV - 3, N_DEV - 2, N_DEV - 1):
                for d in d0[kk]:
                    d.wait_send()

        @pl.when(me == 1)
        def _():
            for kk in (N_DEV - 3, N_DEV - 2, N_DEV - 1):
                for d in d1[kk]:
                    d.wait_send()

        rs_sends = []
        for k in range(1, N_DEV):
            p = (me + k) % N_DEV
            rs = pltpu.make_async_remote_copy(
                src_ref=psend.at[pl.ds(p * RCH, RCH), :],
                dst_ref=rbuf.at[me],
                send_sem=ssem.at[0, k - 1], recv_sem=rsem.at[6, me],
                device_id=(p,), device_id_type=MESH)
            rs.start()
            rs_sends.append(rs)
        rbuf[me] = psend[pl.ds(me * RCH, RCH), :]
        for k in range(1, N_DEV):
            j = (me + k) % N_DEV
            pltpu.make_async_remote_copy(
                src_ref=psend.at[pl.ds(0, RCH), :],
                dst_ref=rbuf.at[j],
                send_sem=ssem.at[0, 0], recv_sem=rsem.at[6, j],
                device_id=(j,), device_id_type=MESH).wait_recv()
        gbuf[me] = jnp.sum(rbuf[...].astype(F32), axis=0).astype(BF)

        ag_sends = []
        for k in range(1, N_DEV):
            p = (me + k) % N_DEV
            ag = pltpu.make_async_remote_copy(
                src_ref=gbuf.at[me], dst_ref=gbuf.at[me],
                send_sem=ssem.at[1, k - 1], recv_sem=rsem.at[7, me],
                device_id=(p,), device_id_type=MESH)
            ag.start()
            ag_sends.append(ag)
        for k in range(1, N_DEV):
            j = (me + k) % N_DEV
            pltpu.make_async_remote_copy(
                src_ref=gbuf.at[me], dst_ref=gbuf.at[j],
                send_sem=ssem.at[1, 0], recv_sem=rsem.at[7, j],
                device_id=(j,), device_id_type=MESH).wait_recv()
        out_ref[...] = gbuf[...].reshape(SQ, SQ).astype(F32)

        for k in range(1, N_DEV):
            for row in (2, 3):
                pltpu.make_async_remote_copy(
                    src_ref=sacc_snd.at[0] if row == 2 else sml_snd.at[0],
                    dst_ref=racc.at[0] if row == 2 else rml.at[0],
                    send_sem=ssem.at[row, k - 1], recv_sem=rsem.at[1 + row, 0],
                    device_id=(0,), device_id_type=MESH).wait_send()
        for d in rs_sends + ag_sends:
            d.wait_send()

    out = pl.pallas_call(
        body,
        out_shape=jax.ShapeDtypeStruct((SQ, SQ), F32),
        in_specs=[
            pl.BlockSpec(memory_space=pltpu.VMEM),
            pl.BlockSpec(memory_space=pltpu.VMEM),
            pl.BlockSpec(memory_space=pl.ANY),
            pl.BlockSpec(memory_space=pl.ANY),
            pl.BlockSpec(memory_space=pltpu.VMEM),
        ],
        out_specs=pl.BlockSpec(memory_space=pltpu.VMEM),
        scratch_shapes=[
            pltpu.VMEM((N_DEV, QG, HD), BF),
            pltpu.VMEM((SKV_SH, HPS, DH), F32),
            pltpu.VMEM((3, SKV_SH, HD), BF),
            pltpu.VMEM((3, SKV_SH, HD), BF),
            pltpu.VMEM((KV1, HD), BF),
            pltpu.VMEM((KV1, HD), BF),
            pltpu.VMEM((N_DEV, QG, HD), BF),
            pltpu.VMEM((N_DEV, QG, DH), F32),
            pltpu.VMEM((N_DEV, QG, HD), BF),
            pltpu.VMEM((N_DEV, QG, DH), F32),
            pltpu.VMEM((SQ, HD), BF),
            pltpu.VMEM((SQ, SQ), BF),
            pltpu.VMEM((N_DEV, RCH, SQ), BF),
            pltpu.VMEM((N_DEV, RCH, SQ), BF),
            pltpu.SemaphoreType.DMA((2,)),
            pltpu.SemaphoreType.DMA((4, N_DEV - 1)),
            pltpu.SemaphoreType.DMA((8, N_DEV)),
        ],
        compiler_params=pltpu.CompilerParams(
            collective_id=0, vmem_limit_bytes=58 * 1024 * 1024),
    )(x, Wq, K_ext, V_ext, Wo)
    return out.reshape(1, SQ, SQ)
